# Initial kernel scaffold; baseline (speedup 1.0000x reference)
import jax
import jax.numpy as jnp
from jax.experimental import pallas as pl


def kernel(x, edge_index, edge_attr, params):
    raise NotImplementedError("write your pallas kernel here")


# speedup vs baseline: 2.9534x; 2.9534x over previous
import functools

import jax
import jax.numpy as jnp
from jax import lax
from jax.experimental import pallas as pl
from jax.experimental.pallas import tpu as pltpu
from jax.experimental.pallas import tpu_sc as plsc

N = 10000
E = 320000
D = 128
NP = 10240
NW = 32
CHUNK = 80
EPW = E // NW
NCHUNKS = EPW // CHUNK
ROWS_PER_TILE = NP // 16


def _edge_body(h_hbm, src_hbm, dst_hbm, attr_hbm, zeros_hbm, out_hbm,
               acc, src_v, dst_v, rows_v, attr_v, sem):
    cid = lax.axis_index("c")
    sid = lax.axis_index("s")
    wid = cid * 16 + sid

    pltpu.sync_copy(zeros_hbm.at[pl.ds(sid * ROWS_PER_TILE, ROWS_PER_TILE)],
                    acc.at[pl.ds(sid * ROWS_PER_TILE, ROWS_PER_TILE)])
    plsc.subcore_barrier()

    base_w = wid * EPW

    def chunk_body(i, carry):
        base = base_w + i * CHUNK
        pltpu.sync_copy(src_hbm.at[pl.ds(base, CHUNK)], src_v)
        pltpu.sync_copy(dst_hbm.at[pl.ds(base, CHUNK)], dst_v)
        pltpu.async_copy(h_hbm.at[src_v], rows_v, sem).wait()
        pltpu.sync_copy(attr_hbm.at[pl.ds(base, CHUNK)], attr_v)

        def row_body(r, c2):
            for cc in range(D // 16):
                sl = pl.ds(cc * 16, 16)
                v = rows_v[r, sl] + attr_v[r, sl]
                rows_v[r, sl] = jnp.maximum(v, 0.0)
            return c2

        lax.fori_loop(0, CHUNK, row_body, 0)
        pltpu.sync_copy(rows_v, acc.at[dst_v], add=True)
        return carry

    lax.fori_loop(0, NCHUNKS, chunk_body, 0)
    plsc.subcore_barrier()

    pltpu.sync_copy(acc.at[pl.ds(sid * ROWS_PER_TILE, ROWS_PER_TILE)],
                    out_hbm.at[cid, pl.ds(sid * ROWS_PER_TILE, ROWS_PER_TILE)])


@jax.jit
def _edge_aggregate(h, src, dst, edge_attr, zeros):
    mesh = plsc.VectorSubcoreMesh(core_axis_name="c", subcore_axis_name="s")
    return pl.kernel(
        _edge_body,
        out_type=jax.ShapeDtypeStruct((2, NP, D), jnp.float32),
        mesh=mesh,
        scratch_types=[
            pltpu.VMEM_SHARED((NP, D), jnp.float32),
            pltpu.VMEM((CHUNK,), jnp.int32),
            pltpu.VMEM((CHUNK,), jnp.int32),
            pltpu.VMEM((CHUNK, D), jnp.float32),
            pltpu.VMEM((CHUNK, D), jnp.float32),
            pltpu.SemaphoreType.DMA,
        ],
    )(h, src, dst, edge_attr, zeros)


def _mlp_body(h_ref, a0_ref, a1_ref, w1_ref, b1_ref, g1_ref, be1_ref,
              w2_ref, b2_ref, g2_ref, be2_ref, out_ref):
    z = h_ref[...] + a0_ref[...] + a1_ref[...]
    z = jnp.dot(z, w1_ref[...], preferred_element_type=jnp.float32) + b1_ref[...]
    mu = jnp.mean(z, axis=0, keepdims=True)
    var = jnp.mean((z - mu) * (z - mu), axis=0, keepdims=True)
    z = g1_ref[...] * (z - mu) / jnp.sqrt(var + 1e-5) + be1_ref[...]
    z = jnp.maximum(z, 0.0)
    z = jnp.dot(z, w2_ref[...], preferred_element_type=jnp.float32) + b2_ref[...]
    z = jnp.maximum(z, 0.0)
    mu = jnp.mean(z, axis=0, keepdims=True)
    var = jnp.mean((z - mu) * (z - mu), axis=0, keepdims=True)
    z = g2_ref[...] * (z - mu) / jnp.sqrt(var + 1e-5) + be2_ref[...]
    out_ref[...] = jnp.maximum(z, 0.0)


@jax.jit
def _mlp(h, a0, a1, w1, b1, g1, be1, w2, b2, g2, be2):
    return pl.pallas_call(
        _mlp_body,
        out_shape=jax.ShapeDtypeStruct((N, D), jnp.float32),
    )(h, a0, a1, w1, b1, g1, be1, w2, b2, g2, be2)


def kernel(x, edge_index, edge_attr, params):
    src = edge_index[0].astype(jnp.int32)
    dst = edge_index[1].astype(jnp.int32)
    zeros = jnp.zeros((NP, D), jnp.float32)
    h = x
    for p in params:
        parts = _edge_aggregate(h, src, dst, edge_attr, zeros)
        a0 = parts[0, :N]
        a1 = parts[1, :N]
        h = _mlp(h, a0, a1,
                 p['W1'], p['b1'].reshape(1, D), p['g1'].reshape(1, D),
                 p['be1'].reshape(1, D),
                 p['W2'], p['b2'].reshape(1, D), p['g2'].reshape(1, D),
                 p['be2'].reshape(1, D))
    return h
